# baseline (speedup 1.0000x reference)
import functools

import jax
import jax.numpy as jnp
from jax import lax
from jax.experimental import pallas as pl
from jax.experimental.pallas import tpu as pltpu
from jax.experimental.pallas import tpu_sc as plsc

VOCAB = 1000000
N_EMBD = 64
BATCH = 4096
HIST = 200

NTOK = BATCH * HIST
NW = 32
TPW = NTOK // NW
G = 640
NCHUNK = TPW // G
NCG = NTOK // G

NCOL = VOCAB // 128
VTAIL = VOCAB - NCOL * 128
COL_ITERS = (NCOL + NW - 1) // NW

_mesh = plsc.VectorSubcoreMesh(core_axis_name="c", subcore_axis_name="s")

_L = 16


@functools.partial(
    pl.kernel,
    out_type=jax.ShapeDtypeStruct((VOCAB // 2, 128), jnp.float32),
    mesh=_mesh,
    scratch_types=[
        pltpu.VMEM((N_EMBD, 128), jnp.float32),
        pltpu.VMEM((N_EMBD, 128), jnp.float32),
    ],
    compiler_params=pltpu.CompilerParams(needs_layout_passes=False),
)
def _pack_table(wt_hbm, tail_hbm, out_hbm, stage_v, pack_v):
    wid = lax.axis_index("s") * 2 + lax.axis_index("c")

    row_idx = [lax.iota(jnp.int32, _L) + _L * jj for jj in range(4)]

    def transpose_cols():
        for q in range(64):
            for h in range(2):
                col = jnp.full((_L,), 2 * q + h, jnp.int32)
                for jj in range(4):
                    vals = plsc.load_gather(stage_v, [row_idx[jj], col])
                    pack_v[q, pl.ds(h * N_EMBD + _L * jj, _L)] = vals

    def body(k, carry):
        t = wid + NW * k

        @pl.when(t < NCOL)
        def _():
            pltpu.sync_copy(wt_hbm.at[:, pl.ds(t * 128, 128)], stage_v)
            transpose_cols()
            pltpu.sync_copy(pack_v, out_hbm.at[pl.ds(t * 64, 64)])

        return carry

    lax.fori_loop(0, COL_ITERS, body, 0)

    @pl.when(wid == 0)
    def _():
        pltpu.sync_copy(tail_hbm, pack_v.at[pl.ds(0, VTAIL // 2)])
        pltpu.sync_copy(
            pack_v.at[pl.ds(0, VTAIL // 2)],
            out_hbm.at[pl.ds(NCOL * 64, VTAIL // 2)],
        )


@functools.partial(
    pl.kernel,
    out_type=jax.ShapeDtypeStruct((NCG, G, 2 * N_EMBD), jnp.float32),
    mesh=_mesh,
    scratch_types=[
        pltpu.VMEM((1, TPW), jnp.int32),
        pltpu.VMEM((G, N_EMBD), jnp.float32),
        pltpu.VMEM((G, N_EMBD), jnp.float32),
        pltpu.SemaphoreType.DMA,
        pltpu.SemaphoreType.DMA,
        pltpu.SemaphoreType.DMA,
        pltpu.SemaphoreType.DMA,
    ],
    compiler_params=pltpu.CompilerParams(use_tc_tiling_on_sc=False),
)
def _emb_lookup(x_hbm, w_hbm, out_hbm, idx_v, rows0, rows1, sg0, sg1, so0, so1):
    wid = lax.axis_index("s") * 2 + lax.axis_index("c")
    base0 = wid * NCHUNK

    pltpu.sync_copy(x_hbm.at[wid], idx_v)

    rows = (rows0, rows1)
    sg = (sg0, sg1)
    so = (so0, so1)

    def fire_gather(c, slot):
        pltpu.async_copy(
            w_hbm.at[idx_v.at[0, pl.ds(c * G, G)]], rows[slot], sg[slot]
        )

    def drain_gather(slot):
        pltpu.make_async_copy(
            w_hbm.at[idx_v.at[0, pl.ds(0, G)]], rows[slot], sg[slot]
        ).wait()

    def out_slice(c):
        return out_hbm.at[base0 + c, :, pl.ds(0, N_EMBD)]

    def drain_out(slot):
        pltpu.make_async_copy(rows[slot], out_slice(0), so[slot]).wait()

    fire_gather(0, 0)

    def body(i, carry):
        for b in range(2):
            c = 2 * i + b
            nxt = 1 - b
            @pl.when(c >= 1)
            def _():
                drain_out(nxt)

            @pl.when(c + 1 < NCHUNK)
            def _():
                fire_gather(c + 1, nxt)

            drain_gather(b)
            pltpu.async_copy(rows[b], out_slice(c), so[b])
        return carry

    lax.fori_loop(0, NCHUNK // 2, body, 0)
    drain_out(1)


def kernel(x, W):
    tail_packed = W[VOCAB - VTAIL :].reshape(VTAIL // 2, 128)
    w_packed = _pack_table(W.T, tail_packed)
    w_flat = w_packed.reshape(VOCAB, N_EMBD)
    x2 = x.astype(jnp.int32).reshape(NW, 1, TPW)
    out = _emb_lookup(x2, w_flat)
    return out.reshape(NTOK, 2 * N_EMBD)[:, :N_EMBD].reshape(
        BATCH, HIST, N_EMBD
    )


# speedup vs baseline: 1.4786x; 1.4786x over previous
import functools

import jax
import jax.numpy as jnp
from jax import lax
from jax.experimental import pallas as pl
from jax.experimental.pallas import tpu as pltpu
from jax.experimental.pallas import tpu_sc as plsc

VOCAB = 1000000
N_EMBD = 64
BATCH = 4096
HIST = 200

NTOK = BATCH * HIST
NW = 32
TPW = NTOK // NW
G = 640
NCHUNK = TPW // G
NCG = NTOK // G

NCOL = VOCAB // 128
VTAIL = VOCAB - NCOL * 128
COL_ITERS = (NCOL + NW - 1) // NW

_mesh = plsc.VectorSubcoreMesh(core_axis_name="c", subcore_axis_name="s")

_L = 16


@functools.partial(
    pl.kernel,
    out_type=jax.ShapeDtypeStruct((VOCAB // 2, 128), jnp.float32),
    mesh=_mesh,
    scratch_types=[
        pltpu.VMEM((N_EMBD, 128), jnp.float32),
        pltpu.VMEM((N_EMBD, 128), jnp.float32),
    ],
    compiler_params=pltpu.CompilerParams(needs_layout_passes=False),
)
def _pack_table(wt_hbm, tail_hbm, out_hbm, stage_v, pack_v):
    wid = lax.axis_index("s") * 2 + lax.axis_index("c")

    row_idx = [lax.iota(jnp.int32, _L) + _L * jj for jj in range(4)]

    def transpose_cols():
        @plsc.parallel_loop(0, 64, unroll=8)
        def _(q):
            for h in range(2):
                col = jnp.full((_L,), 2 * q + h, jnp.int32)
                for jj in range(4):
                    vals = plsc.load_gather(stage_v, [row_idx[jj], col])
                    pack_v[q, pl.ds(h * N_EMBD + _L * jj, _L)] = vals

    def body(k, carry):
        t = wid + NW * k

        @pl.when(t < NCOL)
        def _():
            pltpu.sync_copy(wt_hbm.at[:, pl.ds(t * 128, 128)], stage_v)
            transpose_cols()
            pltpu.sync_copy(pack_v, out_hbm.at[pl.ds(t * 64, 64)])

        return carry

    lax.fori_loop(0, COL_ITERS, body, 0)

    @pl.when(wid == 0)
    def _():
        pltpu.sync_copy(tail_hbm, pack_v.at[pl.ds(0, VTAIL // 2)])
        pltpu.sync_copy(
            pack_v.at[pl.ds(0, VTAIL // 2)],
            out_hbm.at[pl.ds(NCOL * 64, VTAIL // 2)],
        )


@functools.partial(
    pl.kernel,
    out_type=jax.ShapeDtypeStruct((NCG, G, 2 * N_EMBD), jnp.float32),
    mesh=_mesh,
    scratch_types=[
        pltpu.VMEM((1, TPW), jnp.int32),
        pltpu.VMEM((G, N_EMBD), jnp.float32),
        pltpu.VMEM((G, N_EMBD), jnp.float32),
        pltpu.SemaphoreType.DMA,
        pltpu.SemaphoreType.DMA,
        pltpu.SemaphoreType.DMA,
        pltpu.SemaphoreType.DMA,
    ],
    compiler_params=pltpu.CompilerParams(use_tc_tiling_on_sc=False),
)
def _emb_lookup(x_hbm, w_hbm, out_hbm, idx_v, rows0, rows1, sg0, sg1, so0, so1):
    wid = lax.axis_index("s") * 2 + lax.axis_index("c")
    base0 = wid * NCHUNK

    pltpu.sync_copy(x_hbm.at[wid], idx_v)

    rows = (rows0, rows1)
    sg = (sg0, sg1)
    so = (so0, so1)

    def fire_gather(c, slot):
        pltpu.async_copy(
            w_hbm.at[idx_v.at[0, pl.ds(c * G, G)]], rows[slot], sg[slot]
        )

    def drain_gather(slot):
        pltpu.make_async_copy(
            w_hbm.at[idx_v.at[0, pl.ds(0, G)]], rows[slot], sg[slot]
        ).wait()

    def out_slice(c):
        return out_hbm.at[base0 + c, :, pl.ds(0, N_EMBD)]

    def drain_out(slot):
        pltpu.make_async_copy(rows[slot], out_slice(0), so[slot]).wait()

    fire_gather(0, 0)

    def body(i, carry):
        for b in range(2):
            c = 2 * i + b
            nxt = 1 - b
            @pl.when(c >= 1)
            def _():
                drain_out(nxt)

            @pl.when(c + 1 < NCHUNK)
            def _():
                fire_gather(c + 1, nxt)

            drain_gather(b)
            pltpu.async_copy(rows[b], out_slice(c), so[b])
        return carry

    lax.fori_loop(0, NCHUNK // 2, body, 0)
    drain_out(1)


def kernel(x, W):
    tail_packed = W[VOCAB - VTAIL :].reshape(VTAIL // 2, 128)
    w_packed = _pack_table(W.T, tail_packed)
    w_flat = w_packed.reshape(VOCAB, N_EMBD)
    x2 = x.astype(jnp.int32).reshape(NW, 1, TPW)
    out = _emb_lookup(x2, w_flat)
    return out.reshape(NTOK, 2 * N_EMBD)[:, :N_EMBD].reshape(
        BATCH, HIST, N_EMBD
    )


# speedup vs baseline: 2.5243x; 1.7073x over previous
import functools

import jax
import jax.numpy as jnp
from jax import lax
from jax.experimental import pallas as pl
from jax.experimental.pallas import tpu as pltpu
from jax.experimental.pallas import tpu_sc as plsc

VOCAB = 1000000
N_EMBD = 64
BATCH = 4096
HIST = 200

NTOK = BATCH * HIST
NW = 32
TPW = NTOK // NW
G = 800
NCHUNK = TPW // G
NCG = NTOK // G

_mesh = plsc.VectorSubcoreMesh(core_axis_name="c", subcore_axis_name="s")


@functools.partial(
    pl.kernel,
    out_type=jax.ShapeDtypeStruct((NCG, G, 2 * N_EMBD), jnp.float32),
    mesh=_mesh,
    scratch_types=[
        pltpu.VMEM((1, TPW), jnp.int32),
        pltpu.VMEM((G, N_EMBD), jnp.float32),
        pltpu.VMEM((G, N_EMBD), jnp.float32),
        pltpu.SemaphoreType.DMA,
        pltpu.SemaphoreType.DMA,
        pltpu.SemaphoreType.DMA,
        pltpu.SemaphoreType.DMA,
    ],
    compiler_params=pltpu.CompilerParams(use_tc_tiling_on_sc=False),
)
def _emb_lookup(x_hbm, w_hbm, out_hbm, idx_v, rows0, rows1, sg0, sg1, so0, so1):
    wid = lax.axis_index("s") * 2 + lax.axis_index("c")
    base0 = wid * NCHUNK

    pltpu.sync_copy(x_hbm.at[wid], idx_v)

    rows = (rows0, rows1)
    sg = (sg0, sg1)
    so = (so0, so1)

    def fire_gather(c, slot):
        pltpu.async_copy(
            w_hbm.at[idx_v.at[0, pl.ds(c * G, G)]], rows[slot], sg[slot]
        )

    def drain_gather(slot):
        pltpu.make_async_copy(
            w_hbm.at[idx_v.at[0, pl.ds(0, G)]], rows[slot], sg[slot]
        ).wait()

    def out_slice(c):
        return out_hbm.at[base0 + c, :, pl.ds(0, N_EMBD)]

    def drain_out(slot):
        pltpu.make_async_copy(rows[slot], out_slice(0), so[slot]).wait()

    fire_gather(0, 0)

    def body(i, carry):
        for b in range(2):
            c = 2 * i + b
            nxt = 1 - b
            @pl.when(c >= 1)
            def _():
                drain_out(nxt)

            @pl.when(c + 1 < NCHUNK)
            def _():
                fire_gather(c + 1, nxt)

            drain_gather(b)
            pltpu.async_copy(rows[b], out_slice(c), so[b])
        return carry

    lax.fori_loop(0, NCHUNK // 2, body, 0)
    drain_out(1)


def kernel(x, W):
    w_pad = jnp.pad(W, ((0, 0), (0, 128 - N_EMBD)))
    w_flat = w_pad.reshape(2 * VOCAB, N_EMBD)
    x2 = (x.astype(jnp.int32) * 2).reshape(NW, 1, TPW)
    out = _emb_lookup(x2, w_flat)
    return out.reshape(NTOK, 2 * N_EMBD)[:, :N_EMBD].reshape(
        BATCH, HIST, N_EMBD
    )


# speedup vs baseline: 2.5270x; 1.0011x over previous
import functools

import jax
import jax.numpy as jnp
from jax import lax
from jax.experimental import pallas as pl
from jax.experimental.pallas import tpu as pltpu
from jax.experimental.pallas import tpu_sc as plsc

VOCAB = 1000000
N_EMBD = 64
BATCH = 4096
HIST = 200

NTOK = BATCH * HIST
NW = 32
TPW = NTOK // NW
G = 640
NCHUNK = TPW // G
NCG = NTOK // G

_mesh = plsc.VectorSubcoreMesh(core_axis_name="c", subcore_axis_name="s")


@functools.partial(
    pl.kernel,
    out_type=jax.ShapeDtypeStruct((NCG, G, 2 * N_EMBD), jnp.float32),
    mesh=_mesh,
    scratch_types=[
        pltpu.VMEM((1, TPW), jnp.int32),
        pltpu.VMEM((G, N_EMBD), jnp.float32),
        pltpu.VMEM((G, N_EMBD), jnp.float32),
        pltpu.SemaphoreType.DMA,
        pltpu.SemaphoreType.DMA,
        pltpu.SemaphoreType.DMA,
        pltpu.SemaphoreType.DMA,
    ],
    compiler_params=pltpu.CompilerParams(use_tc_tiling_on_sc=False),
)
def _emb_lookup(x_hbm, w_hbm, out_hbm, idx_v, rows0, rows1, sg0, sg1, so0, so1):
    wid = lax.axis_index("s") * 2 + lax.axis_index("c")
    base0 = wid * NCHUNK

    pltpu.sync_copy(x_hbm.at[wid], idx_v)

    rows = (rows0, rows1)
    sg = (sg0, sg1)
    so = (so0, so1)

    def fire_gather(c, slot):
        pltpu.async_copy(
            w_hbm.at[idx_v.at[0, pl.ds(c * G, G)]], rows[slot], sg[slot]
        )

    def drain_gather(slot):
        pltpu.make_async_copy(
            w_hbm.at[idx_v.at[0, pl.ds(0, G)]], rows[slot], sg[slot]
        ).wait()

    def out_slice(c):
        return out_hbm.at[base0 + c, :, pl.ds(0, N_EMBD)]

    def drain_out(slot):
        pltpu.make_async_copy(rows[slot], out_slice(0), so[slot]).wait()

    fire_gather(0, 0)

    def body(i, carry):
        for b in range(2):
            c = 2 * i + b
            nxt = 1 - b
            @pl.when(c >= 1)
            def _():
                drain_out(nxt)

            @pl.when(c + 1 < NCHUNK)
            def _():
                fire_gather(c + 1, nxt)

            drain_gather(b)
            pltpu.async_copy(rows[b], out_slice(c), so[b])
        return carry

    lax.fori_loop(0, NCHUNK // 2, body, 0)
    drain_out(1)


def kernel(x, W):
    w_pad = jnp.pad(W, ((0, 0), (0, 128 - N_EMBD)))
    w_flat = w_pad.reshape(2 * VOCAB, N_EMBD)
    x2 = (x.astype(jnp.int32) * 2).reshape(NW, 1, TPW)
    out = _emb_lookup(x2, w_flat)
    return out.reshape(NTOK, 2 * N_EMBD)[:, :N_EMBD].reshape(
        BATCH, HIST, N_EMBD
    )
